# baseline (speedup 1.0000x reference)
import jax
import jax.numpy as jnp
from jax import lax
from jax.experimental import pallas as pl
from jax.experimental.pallas import tpu as pltpu
from jax.experimental.pallas import tpu_sc as plsc

_NUM_OPS = 8
_NUM_ON = 2
_LANES = 16
_TILE = 4096


def _permute(v, perm):
    return lax.gather(
        v,
        perm[:, None],
        lax.GatherDimensionNumbers(
            offset_dims=(), collapsed_slice_dims=(0,), start_index_map=(0,)),
        (1,),
        mode=lax.GatherScatterMode.PROMISE_IN_BOUNDS,
    )


def _allmax(v, ids):
    for s in (1, 2, 4, 8):
        v = jnp.maximum(v, _permute(v, ids ^ s))
    return v


def _allmin(v, ids):
    for s in (1, 2, 4, 8):
        v = jnp.minimum(v, _permute(v, ids ^ s))
    return v


def _argmax_first(p, ids):
    return _allmin(jnp.where(p == _allmax(p, ids), ids, _LANES), ids)


def _sc_routing(logits_hbm, z_hbm, tabs_hbm, sb_hbm,
                l_v, z_v, tabs_v, s_v, b_v):
    D = tabs_hbm.shape[0] // (2 * _NUM_OPS)
    cid = lax.axis_index("c")
    sid = lax.axis_index("s")
    wid = sid * 2 + cid

    pltpu.sync_copy(logits_hbm, l_v)
    pltpu.sync_copy(z_hbm, z_v)
    pltpu.sync_copy(tabs_hbm, tabs_v)
    p = l_v[...] + z_v[...]
    ids = lax.iota(jnp.int32, _LANES)
    i0 = _argmax_first(p, ids)
    p2 = jnp.where(ids == i0, -jnp.inf, p)
    i1 = _argmax_first(p2, ids)

    n_chunks = D // _LANES
    for r in range(2):
        chunk = wid + 32 * r

        @pl.when(chunk < n_chunks)
        def _do_chunk():
            col = chunk * _LANES
            s = jnp.zeros((_LANES,), jnp.float32)
            b = jnp.zeros((_LANES,), jnp.float32)
            for e in range(_NUM_OPS):
                on_e = (i0 == e) | (i1 == e)
                s = s + jnp.where(on_e, tabs_v[pl.ds(e * D + col, _LANES)], 0.0)
                b = b + jnp.where(
                    on_e, tabs_v[pl.ds((_NUM_OPS + e) * D + col, _LANES)], 0.0)
            s_v[...] = s
            b_v[...] = b
            pltpu.sync_copy(s_v, sb_hbm.at[pl.ds(col, _LANES)])
            pltpu.sync_copy(b_v, sb_hbm.at[pl.ds(D + col, _LANES)])


def _apply_kernel(sb_ref, x_ref, out_ref):
    out_ref[...] = x_ref[...] * sb_ref[0:1, :] + sb_ref[1:2, :]


def kernel(x, logits, scales, biases):
    T, D = x.shape
    gkey = jax.random.fold_in(jax.random.key(0), 123)
    u = jax.random.uniform(gkey, logits.shape, minval=1e-20, maxval=1.0)
    z = -jnp.log(-jnp.log(u))

    neg_inf = jnp.float32(-jnp.inf)
    l16 = jnp.full((_LANES,), neg_inf, jnp.float32).at[:_NUM_OPS].set(logits)
    z16 = jnp.zeros((_LANES,), jnp.float32).at[:_NUM_OPS].set(z)

    tabs_flat = jnp.concatenate([scales.reshape(-1), biases.reshape(-1)])

    mesh = plsc.VectorSubcoreMesh(core_axis_name="c", subcore_axis_name="s")
    routing = pl.kernel(
        _sc_routing,
        mesh=mesh,
        out_type=jax.ShapeDtypeStruct((2 * D,), jnp.float32),
        scratch_types=[
            pltpu.VMEM((_LANES,), jnp.float32),
            pltpu.VMEM((_LANES,), jnp.float32),
            pltpu.VMEM((2 * _NUM_OPS * D,), jnp.float32),
            pltpu.VMEM((_LANES,), jnp.float32),
            pltpu.VMEM((_LANES,), jnp.float32),
        ],
    )
    sb = routing(l16, z16, tabs_flat).reshape(2, D)

    grid = (T // _TILE,)
    out = pl.pallas_call(
        _apply_kernel,
        grid=grid,
        in_specs=[
            pl.BlockSpec((2, D), lambda i: (0, 0)),
            pl.BlockSpec((_TILE, D), lambda i: (i, 0)),
        ],
        out_specs=pl.BlockSpec((_TILE, D), lambda i: (i, 0)),
        out_shape=jax.ShapeDtypeStruct((T, D), x.dtype),
        compiler_params=pltpu.CompilerParams(
            dimension_semantics=("arbitrary",),
            vmem_limit_bytes=100 * 1024 * 1024,
        ),
    )(sb, x)
    return out


# speedup vs baseline: 1.0173x; 1.0173x over previous
import jax
import jax.numpy as jnp
from jax import lax
from jax.experimental import pallas as pl
from jax.experimental.pallas import tpu as pltpu
from jax.experimental.pallas import tpu_sc as plsc

_NUM_OPS = 8
_NUM_ON = 2
_LANES = 16
_TILE = 4096


def _permute(v, perm):
    return lax.gather(
        v,
        perm[:, None],
        lax.GatherDimensionNumbers(
            offset_dims=(), collapsed_slice_dims=(0,), start_index_map=(0,)),
        (1,),
        mode=lax.GatherScatterMode.PROMISE_IN_BOUNDS,
    )


def _allmax(v, ids):
    for s in (1, 2, 4, 8):
        v = jnp.maximum(v, _permute(v, ids ^ s))
    return v


def _allmin(v, ids):
    for s in (1, 2, 4, 8):
        v = jnp.minimum(v, _permute(v, ids ^ s))
    return v


def _argmax_first(p, ids):
    return _allmin(jnp.where(p == _allmax(p, ids), ids, _LANES), ids)


def _sc_routing(lz_hbm, tabs_hbm, sb_hbm,
                lz_v, tabs_v, acc_v, sem1, sem2):
    D = tabs_hbm.shape[0] // (2 * _NUM_OPS)
    span = D // 16
    cid = lax.axis_index("c")
    sid = lax.axis_index("s")
    wid = sid * 2 + cid

    c1 = pltpu.make_async_copy(lz_hbm, lz_v, sem1)
    c2 = pltpu.make_async_copy(tabs_hbm, tabs_v, sem2)
    c1.start()
    c2.start()
    c1.wait()
    p = lz_v[pl.ds(0, _LANES)] + lz_v[pl.ds(_LANES, _LANES)]
    ids = lax.iota(jnp.int32, _LANES)
    i0 = _argmax_first(p, ids)
    p2 = jnp.where(ids == i0, -jnp.inf, p)
    i1 = _argmax_first(p2, ids)
    c2.wait()

    half = (wid >= 16).astype(jnp.int32)
    tab_base = half * (_NUM_OPS * D)
    col0 = (wid - half * 16) * span
    for k in range(span // _LANES):
        a = jnp.zeros((_LANES,), jnp.float32)
        for e in range(_NUM_OPS):
            on_e = (i0 == e) | (i1 == e)
            a = a + jnp.where(
                on_e,
                tabs_v[pl.ds(tab_base + e * D + col0 + k * _LANES, _LANES)],
                0.0)
        acc_v[pl.ds(k * _LANES, _LANES)] = a
    pltpu.sync_copy(acc_v, sb_hbm.at[pl.ds(half * D + col0, span)])


def _apply_kernel(sb_ref, x_ref, out_ref):
    out_ref[...] = x_ref[...] * sb_ref[0:1, :] + sb_ref[1:2, :]


def kernel(x, logits, scales, biases):
    T, D = x.shape
    gkey = jax.random.fold_in(jax.random.key(0), 123)
    u = jax.random.uniform(gkey, logits.shape, minval=1e-20, maxval=1.0)
    z = -jnp.log(-jnp.log(u))

    neg_inf = jnp.float32(-jnp.inf)
    lz = jnp.concatenate([
        jnp.pad(logits, (0, _LANES - _NUM_OPS), constant_values=neg_inf),
        jnp.pad(z, (0, _LANES - _NUM_OPS)),
    ])
    tabs_flat = jnp.concatenate([scales.reshape(-1), biases.reshape(-1)])

    mesh = plsc.VectorSubcoreMesh(core_axis_name="c", subcore_axis_name="s")
    routing = pl.kernel(
        _sc_routing,
        mesh=mesh,
        out_type=jax.ShapeDtypeStruct((2 * D,), jnp.float32),
        scratch_types=[
            pltpu.VMEM((2 * _LANES,), jnp.float32),
            pltpu.VMEM((2 * _NUM_OPS * D,), jnp.float32),
            pltpu.VMEM((D // 16,), jnp.float32),
            pltpu.SemaphoreType.DMA,
            pltpu.SemaphoreType.DMA,
        ],
    )
    sb = routing(lz, tabs_flat).reshape(2, D)

    grid = (T // _TILE,)
    out = pl.pallas_call(
        _apply_kernel,
        grid=grid,
        in_specs=[
            pl.BlockSpec((2, D), lambda i: (0, 0)),
            pl.BlockSpec((_TILE, D), lambda i: (i, 0)),
        ],
        out_specs=pl.BlockSpec((_TILE, D), lambda i: (i, 0)),
        out_shape=jax.ShapeDtypeStruct((T, D), x.dtype),
        compiler_params=pltpu.CompilerParams(
            dimension_semantics=("arbitrary",),
            vmem_limit_bytes=100 * 1024 * 1024,
        ),
    )(sb, x)
    return out


# speedup vs baseline: 1.4198x; 1.3957x over previous
import jax
import jax.numpy as jnp
from jax.experimental import pallas as pl
from jax.experimental.pallas import tpu as pltpu

_NUM_OPS = 8
_NUM_ON = 2
_TILE = 4096


def _mix_kernel(logits_ref, z_ref, scales_ref, biases_ref, x_ref, out_ref,
                sb_ref):
    @pl.when(pl.program_id(0) == 0)
    def _routing():
        logits = logits_ref[...]
        z = z_ref[...]
        p = logits + z
        ids = jax.lax.broadcasted_iota(jnp.int32, (1, _NUM_OPS), 1)
        max0 = jnp.max(p, axis=1, keepdims=True)
        i0 = jnp.min(jnp.where(p == max0, ids, _NUM_OPS), axis=1, keepdims=True)
        p2 = jnp.where(ids == i0, -jnp.inf, p)
        max1 = jnp.max(p2, axis=1, keepdims=True)
        i1 = jnp.min(jnp.where(p2 == max1, ids, _NUM_OPS), axis=1, keepdims=True)
        i0s = i0[0, 0]
        i1s = i1[0, 0]
        rows = jax.lax.broadcasted_iota(jnp.int32, (_NUM_OPS, 1), 0)
        sel = (rows == i0s) | (rows == i1s)
        S = jnp.sum(jnp.where(sel, scales_ref[...], 0.0), axis=0, keepdims=True)
        B = jnp.sum(jnp.where(sel, biases_ref[...], 0.0), axis=0, keepdims=True)
        sb_ref[0:1, :] = S
        sb_ref[1:2, :] = B

    S = sb_ref[0:1, :]
    B = sb_ref[1:2, :]
    out_ref[...] = x_ref[...] * S + B


def kernel(x, logits, scales, biases):
    T, D = x.shape
    gkey = jax.random.fold_in(jax.random.key(0), 123)
    u = jax.random.uniform(gkey, logits.shape, minval=1e-20, maxval=1.0)
    z = -jnp.log(-jnp.log(u))

    logits2 = logits.reshape(1, _NUM_OPS)
    z2 = z.reshape(1, _NUM_OPS)

    grid = (T // _TILE,)
    out = pl.pallas_call(
        _mix_kernel,
        grid=grid,
        in_specs=[
            pl.BlockSpec((1, _NUM_OPS), lambda i: (0, 0)),
            pl.BlockSpec((1, _NUM_OPS), lambda i: (0, 0)),
            pl.BlockSpec((_NUM_OPS, D), lambda i: (0, 0)),
            pl.BlockSpec((_NUM_OPS, D), lambda i: (0, 0)),
            pl.BlockSpec((_TILE, D), lambda i: (i, 0)),
        ],
        out_specs=pl.BlockSpec((_TILE, D), lambda i: (i, 0)),
        out_shape=jax.ShapeDtypeStruct((T, D), x.dtype),
        scratch_shapes=[pltpu.VMEM((2, D), jnp.float32)],
        compiler_params=pltpu.CompilerParams(
            dimension_semantics=("arbitrary",),
            vmem_limit_bytes=100 * 1024 * 1024,
        ),
    )(logits2, z2, scales, biases, x)
    return out
